# baseline (speedup 1.0000x reference)
import jax
import jax.numpy as jnp
from jax.experimental import pallas as pl

_TILE = 8192


def _fused_mlp_kernel(f_ref, c_ref, w1_ref, b1_ref, w2_ref, b2_ref, q_ref,
                      o_ref):
    x = jnp.concatenate([f_ref[0], c_ref[0]], axis=-1)
    h = jnp.dot(x, w1_ref[...], preferred_element_type=jnp.float32)
    h = jnp.maximum(h + b1_ref[...], 0.0)
    h = jnp.dot(h, w2_ref[...], preferred_element_type=jnp.float32)
    h = jnp.maximum(h + b2_ref[...], 0.0)
    o_ref[...] = jax.lax.dot_general(
        q_ref[...], h,
        dimension_numbers=(((1,), (1,)), ((), ())),
        preferred_element_type=jnp.float32,
    )


def kernel(coord, feat, batch, w1, b1, w2, b2, queries):
    n, d_feat = feat.shape
    d_coord = coord.shape[1]
    d_in, embed = w1.shape
    k = queries.shape[0]
    tile = _TILE if n % _TILE == 0 else n
    grid = n // tile
    feat3 = feat.reshape(grid, tile, d_feat)
    coord3 = coord.reshape(grid, tile, d_coord)

    masks = pl.pallas_call(
        _fused_mlp_kernel,
        grid=(grid,),
        in_specs=[
            pl.BlockSpec((1, tile, d_feat), lambda i: (i, 0, 0)),
            pl.BlockSpec((1, tile, d_coord), lambda i: (i, 0, 0)),
            pl.BlockSpec((d_in, embed), lambda i: (0, 0)),
            pl.BlockSpec((1, embed), lambda i: (0, 0)),
            pl.BlockSpec((embed, embed), lambda i: (0, 0)),
            pl.BlockSpec((1, embed), lambda i: (0, 0)),
            pl.BlockSpec((k, embed), lambda i: (0, 0)),
        ],
        out_specs=pl.BlockSpec((k, tile), lambda i: (0, i)),
        out_shape=jax.ShapeDtypeStruct((k, n), jnp.float32),
    )(feat3, coord3, w1, b1[None, :], w2, b2[None, :], queries)
    return masks, batch


# speedup vs baseline: 1.6175x; 1.6175x over previous
import jax
import jax.numpy as jnp
from jax.experimental import pallas as pl

_TILE = 8192


def _fused_mlp_kernel(xt_ref, w1_ref, b1_ref, w2_ref, b2_ref, q_ref, o_ref):
    h = jax.lax.dot_general(
        w1_ref[...], xt_ref[...],
        dimension_numbers=(((0,), (0,)), ((), ())),
        preferred_element_type=jnp.float32,
    )
    h = jnp.maximum(h + b1_ref[...], 0.0)
    h = jax.lax.dot_general(
        w2_ref[...], h,
        dimension_numbers=(((0,), (0,)), ((), ())),
        preferred_element_type=jnp.float32,
    )
    h = jnp.maximum(h + b2_ref[...], 0.0)
    o_ref[...] = jax.lax.dot_general(
        q_ref[...], h,
        dimension_numbers=(((1,), (0,)), ((), ())),
        preferred_element_type=jnp.float32,
    )


def kernel(coord, feat, batch, w1, b1, w2, b2, queries):
    xt = jnp.concatenate([feat, coord], axis=-1).T
    d_in, n = xt.shape
    embed = w1.shape[1]
    k = queries.shape[0]
    tile = _TILE if n % _TILE == 0 else n
    grid = n // tile

    masks = pl.pallas_call(
        _fused_mlp_kernel,
        grid=(grid,),
        in_specs=[
            pl.BlockSpec((d_in, tile), lambda i: (0, i)),
            pl.BlockSpec((d_in, embed), lambda i: (0, 0)),
            pl.BlockSpec((embed, 1), lambda i: (0, 0)),
            pl.BlockSpec((embed, embed), lambda i: (0, 0)),
            pl.BlockSpec((embed, 1), lambda i: (0, 0)),
            pl.BlockSpec((k, embed), lambda i: (0, 0)),
        ],
        out_specs=pl.BlockSpec((k, tile), lambda i: (0, i)),
        out_shape=jax.ShapeDtypeStruct((k, n), jnp.float32),
    )(xt, w1, b1[:, None], w2, b2[:, None], queries)
    return masks, batch


# speedup vs baseline: 1.8919x; 1.1697x over previous
import jax
import jax.numpy as jnp
from jax.experimental import pallas as pl

_TILE = 8192


def _fused_mlp_kernel(xt_ref, w1_ref, b1_ref, w2_ref, b2_ref, q_ref, o_ref):
    h = jax.lax.dot_general(
        xt_ref[...], w1_ref[...],
        dimension_numbers=(((0,), (0,)), ((), ())),
        preferred_element_type=jnp.float32,
    )
    h = jnp.maximum(h + b1_ref[...], 0.0)
    h = jnp.dot(h, w2_ref[...], preferred_element_type=jnp.float32)
    h = jnp.maximum(h + b2_ref[...], 0.0)
    o_ref[...] = jax.lax.dot_general(
        q_ref[...], h,
        dimension_numbers=(((1,), (1,)), ((), ())),
        preferred_element_type=jnp.float32,
    )


def kernel(coord, feat, batch, w1, b1, w2, b2, queries):
    xt = jnp.concatenate([feat, coord], axis=-1).T
    d_in, n = xt.shape
    embed = w1.shape[1]
    k = queries.shape[0]
    tile = _TILE if n % _TILE == 0 else n
    grid = n // tile

    masks = pl.pallas_call(
        _fused_mlp_kernel,
        grid=(grid,),
        in_specs=[
            pl.BlockSpec((d_in, tile), lambda i: (0, i)),
            pl.BlockSpec((d_in, embed), lambda i: (0, 0)),
            pl.BlockSpec((1, embed), lambda i: (0, 0)),
            pl.BlockSpec((embed, embed), lambda i: (0, 0)),
            pl.BlockSpec((1, embed), lambda i: (0, 0)),
            pl.BlockSpec((k, embed), lambda i: (0, 0)),
        ],
        out_specs=pl.BlockSpec((k, tile), lambda i: (0, i)),
        out_shape=jax.ShapeDtypeStruct((k, n), jnp.float32),
    )(xt, w1, b1[None, :], w2, b2[None, :], queries)
    return masks, batch
